# baseline (speedup 1.0000x reference)
import functools

import jax
import jax.numpy as jnp
from jax import lax
from jax.experimental import pallas as pl
from jax.experimental.pallas import tpu as pltpu
from jax.experimental.pallas import tpu_sc as plsc

HIDDEN = 64
CHUNK = 128
BN = 2048
BNP = BN // 2


def _pack_table(table_t):
    v = table_t.shape[1]
    grid = -(-v // BN)
    vp = grid * BNP

    def pack_k(a_ref, o_ref):
        y = jnp.transpose(a_ref[...], (1, 0))
        o_ref[...] = jnp.concatenate([y[:BNP], y[BNP:]], axis=1)

    return pl.pallas_call(
        pack_k,
        grid=(grid,),
        in_specs=[pl.BlockSpec((64, BN), lambda i: (0, i))],
        out_specs=pl.BlockSpec((BNP, 128), lambda i: (i, 0)),
        out_shape=jax.ShapeDtypeStruct((vp, 128), jnp.float32),
    )(table_t)


def _remap_indices(idx):
    i_blk = idx // BN
    q = idx % BN
    return i_blk * BN + jnp.where(q < BNP, 2 * q, 2 * (q - BNP) + 1)


def _gather_rows(idx2d, dst2d, table):
    n_chunks, chunk = idx2d.shape
    total = n_chunks * chunk
    info = plsc.get_sparse_core_info()
    nw = info.num_cores * info.num_subcores
    chunks_per_w = n_chunks // nw

    mesh = plsc.VectorSubcoreMesh(core_axis_name="c", subcore_axis_name="s")

    grp = 4
    n_groups = chunks_per_w // grp

    @functools.partial(
        pl.kernel,
        mesh=mesh,
        out_type=jax.ShapeDtypeStruct((total, HIDDEN), jnp.float32),
        scratch_types=[
            pltpu.VMEM((chunks_per_w, chunk), jnp.int32),
            pltpu.VMEM((chunks_per_w, chunk), jnp.int32),
            pltpu.VMEM((2 * grp, chunk, HIDDEN), jnp.float32),
            pltpu.SemaphoreType.DMA,
            pltpu.SemaphoreType.DMA,
        ],
        compiler_params=pltpu.CompilerParams(use_tc_tiling_on_sc=False),
    )
    def gather_k(idx_hbm, dst_hbm, table_hbm, out_hbm, idx_v, dst_v, rows_v, gsem, osem):
        wid = lax.axis_index("s") * info.num_cores + lax.axis_index("c")
        cbase = wid * chunks_per_w
        pltpu.sync_copy(idx_hbm.at[pl.ds(cbase, chunks_per_w)], idx_v)
        pltpu.sync_copy(dst_hbm.at[pl.ds(cbase, chunks_per_w)], dst_v)

        def fire_group(g, sb):
            for i in range(grp):
                pltpu.async_copy(
                    table_hbm.at[idx_v.at[g * grp + i]], rows_v.at[sb + i], gsem
                )

        def drain(sem, n):
            for _ in range(n):
                pltpu.make_async_copy(
                    table_hbm.at[pl.ds(0, chunk)], rows_v.at[0], sem
                ).wait()

        fire_group(0, 0)

        def body(g, carry):
            sb = (g % 2) * grp
            nsb = grp - sb
            drain(gsem, grp)

            @pl.when(g >= 1)
            def _():
                drain(osem, grp)

            @pl.when(g + 1 < n_groups)
            def _():
                fire_group(g + 1, nsb)

            for i in range(grp):
                pltpu.async_copy(
                    rows_v.at[sb + i],
                    out_hbm.at[dst_v.at[g * grp + i]],
                    osem,
                )
            return carry

        lax.fori_loop(0, n_groups, body, 0)
        drain(osem, grp)

    return gather_k(idx2d, dst2d, table)


def _project(g2, W, b, bsz, npair):
    bm = 2048
    nb = bsz // bm

    def mm_k(x_ref, w_ref, b_ref, o_ref):
        t = pl.program_id(1)

        @pl.when(t == 0)
        def _():
            o_ref[...] = jnp.broadcast_to(b_ref[...], o_ref.shape)

        o_ref[...] += jnp.dot(
            x_ref[...], w_ref[...], preferred_element_type=jnp.float32
        )

    return pl.pallas_call(
        mm_k,
        grid=(nb, npair),
        in_specs=[
            pl.BlockSpec((bm, 2 * HIDDEN), lambda i, t: (t * nb + i, 0)),
            pl.BlockSpec((2 * HIDDEN, HIDDEN), lambda i, t: (t, 0)),
            pl.BlockSpec((1, HIDDEN), lambda i, t: (0, 0)),
        ],
        out_specs=pl.BlockSpec((bm, HIDDEN), lambda i, t: (i, 0)),
        out_shape=jax.ShapeDtypeStruct((bsz, HIDDEN), jnp.float32),
    )(g2, W, b.reshape(1, HIDDEN))


def kernel(int_vals, table, W, b):
    bsz, s = int_vals.shape
    npair = s // 2
    packed = _pack_table(table.T)
    table_lin = packed.reshape(2 * packed.shape[0], HIDDEN)
    idx2d = _remap_indices(int_vals).reshape(-1, CHUNK)
    bb = jnp.arange(bsz, dtype=jnp.int32)[:, None]
    ss = jnp.arange(s, dtype=jnp.int32)[None, :]
    dst2d = (2 * ((ss // 2) * bsz + bb) + ss % 2).reshape(-1, CHUNK)
    gathered = _gather_rows(idx2d, dst2d, table_lin)
    g2 = gathered.reshape(npair * bsz, 2 * HIDDEN)
    return _project(g2, W, b, bsz, npair)


# speedup vs baseline: 1.5401x; 1.5401x over previous
import functools

import jax
import jax.numpy as jnp
from jax import lax
from jax.experimental import pallas as pl
from jax.experimental.pallas import tpu as pltpu
from jax.experimental.pallas import tpu_sc as plsc

HIDDEN = 64
CHUNK = 128
BN = 16384
BNP = BN // 2


def _pack_table(table_t):
    v = table_t.shape[1]
    grid = -(-v // BN)
    vp = grid * BNP

    def pack_k(a_ref, o_ref):
        y = jnp.transpose(a_ref[...], (1, 0))
        o_ref[...] = jnp.concatenate([y[:BNP], y[BNP:]], axis=1)

    return pl.pallas_call(
        pack_k,
        grid=(grid,),
        in_specs=[pl.BlockSpec((64, BN), lambda i: (0, i))],
        out_specs=pl.BlockSpec((BNP, 128), lambda i: (i, 0)),
        out_shape=jax.ShapeDtypeStruct((vp, 128), jnp.float32),
    )(table_t)


def _remap_indices(idx):
    i_blk = idx // BN
    q = idx % BN
    return i_blk * BN + jnp.where(q < BNP, 2 * q, 2 * (q - BNP) + 1)


def _gather_rows(idx2d, dst2d, table):
    n_chunks, chunk = idx2d.shape
    total = n_chunks * chunk
    info = plsc.get_sparse_core_info()
    nw = info.num_cores * info.num_subcores
    chunks_per_w = n_chunks // nw

    mesh = plsc.VectorSubcoreMesh(core_axis_name="c", subcore_axis_name="s")

    grp = 4
    n_groups = chunks_per_w // grp

    @functools.partial(
        pl.kernel,
        mesh=mesh,
        out_type=jax.ShapeDtypeStruct((total, HIDDEN), jnp.float32),
        scratch_types=[
            pltpu.VMEM((chunks_per_w, chunk), jnp.int32),
            pltpu.VMEM((chunks_per_w, chunk), jnp.int32),
            pltpu.VMEM((2 * grp, chunk, HIDDEN), jnp.float32),
            pltpu.SemaphoreType.DMA,
            pltpu.SemaphoreType.DMA,
        ],
        compiler_params=pltpu.CompilerParams(use_tc_tiling_on_sc=False),
    )
    def gather_k(idx_hbm, dst_hbm, table_hbm, out_hbm, idx_v, dst_v, rows_v, gsem, osem):
        wid = lax.axis_index("s") * info.num_cores + lax.axis_index("c")
        cbase = wid * chunks_per_w
        pltpu.sync_copy(idx_hbm.at[pl.ds(cbase, chunks_per_w)], idx_v)
        pltpu.sync_copy(dst_hbm.at[pl.ds(cbase, chunks_per_w)], dst_v)

        def fire_group(g, sb):
            for i in range(grp):
                pltpu.async_copy(
                    table_hbm.at[idx_v.at[g * grp + i]], rows_v.at[sb + i], gsem
                )

        def drain(sem, n):
            for _ in range(n):
                pltpu.make_async_copy(
                    table_hbm.at[pl.ds(0, chunk)], rows_v.at[0], sem
                ).wait()

        fire_group(0, 0)

        def body(g, carry):
            sb = (g % 2) * grp
            nsb = grp - sb
            drain(gsem, grp)

            @pl.when(g >= 1)
            def _():
                drain(osem, grp)

            @pl.when(g + 1 < n_groups)
            def _():
                fire_group(g + 1, nsb)

            for i in range(grp):
                pltpu.async_copy(
                    rows_v.at[sb + i],
                    out_hbm.at[dst_v.at[g * grp + i]],
                    osem,
                )
            return carry

        lax.fori_loop(0, n_groups, body, 0)
        drain(osem, grp)

    return gather_k(idx2d, dst2d, table)


def _project(g2, W, b, bsz, npair):
    bm = 2048
    nb = bsz // bm

    def mm_k(x_ref, w_ref, b_ref, o_ref):
        t = pl.program_id(1)

        @pl.when(t == 0)
        def _():
            o_ref[...] = jnp.broadcast_to(b_ref[...], o_ref.shape)

        o_ref[...] += jnp.dot(
            x_ref[...], w_ref[...], preferred_element_type=jnp.float32
        )

    return pl.pallas_call(
        mm_k,
        grid=(nb, npair),
        in_specs=[
            pl.BlockSpec((bm, 2 * HIDDEN), lambda i, t: (t * nb + i, 0)),
            pl.BlockSpec((2 * HIDDEN, HIDDEN), lambda i, t: (t, 0)),
            pl.BlockSpec((1, HIDDEN), lambda i, t: (0, 0)),
        ],
        out_specs=pl.BlockSpec((bm, HIDDEN), lambda i, t: (i, 0)),
        out_shape=jax.ShapeDtypeStruct((bsz, HIDDEN), jnp.float32),
    )(g2, W, b.reshape(1, HIDDEN))


def kernel(int_vals, table, W, b):
    bsz, s = int_vals.shape
    npair = s // 2
    packed = _pack_table(table.T)
    table_lin = packed.reshape(2 * packed.shape[0], HIDDEN)
    idx2d = _remap_indices(int_vals).reshape(-1, CHUNK)
    bb = jnp.arange(bsz, dtype=jnp.int32)[:, None]
    ss = jnp.arange(s, dtype=jnp.int32)[None, :]
    dst2d = (2 * ((ss // 2) * bsz + bb) + ss % 2).reshape(-1, CHUNK)
    gathered = _gather_rows(idx2d, dst2d, table_lin)
    g2 = gathered.reshape(npair * bsz, 2 * HIDDEN)
    return _project(g2, W, b, bsz, npair)


# speedup vs baseline: 1.6038x; 1.0414x over previous
import functools

import jax
import jax.numpy as jnp
from jax import lax
from jax.experimental import pallas as pl
from jax.experimental.pallas import tpu as pltpu
from jax.experimental.pallas import tpu_sc as plsc

HIDDEN = 64
CHUNK = 128
BN = 32768
BNP = BN // 2


def _pack_table(table_t):
    v = table_t.shape[1]
    grid = -(-v // BN)
    vp = grid * BNP

    def pack_k(a_ref, o_ref):
        y = jnp.transpose(a_ref[...], (1, 0))
        o_ref[...] = jnp.concatenate([y[:BNP], y[BNP:]], axis=1)

    return pl.pallas_call(
        pack_k,
        grid=(grid,),
        in_specs=[pl.BlockSpec((64, BN), lambda i: (0, i))],
        out_specs=pl.BlockSpec((BNP, 128), lambda i: (i, 0)),
        out_shape=jax.ShapeDtypeStruct((vp, 128), jnp.float32),
    )(table_t)


def _remap_indices(idx):
    i_blk = idx // BN
    q = idx % BN
    return i_blk * BN + jnp.where(q < BNP, 2 * q, 2 * (q - BNP) + 1)


def _gather_rows(idx2d, dst2d, table):
    n_chunks, chunk = idx2d.shape
    total = n_chunks * chunk
    info = plsc.get_sparse_core_info()
    nw = info.num_cores * info.num_subcores
    chunks_per_w = n_chunks // nw

    mesh = plsc.VectorSubcoreMesh(core_axis_name="c", subcore_axis_name="s")

    grp = 4
    n_groups = chunks_per_w // grp

    @functools.partial(
        pl.kernel,
        mesh=mesh,
        out_type=jax.ShapeDtypeStruct((total, HIDDEN), jnp.float32),
        scratch_types=[
            pltpu.VMEM((chunks_per_w, chunk), jnp.int32),
            pltpu.VMEM((chunks_per_w, chunk), jnp.int32),
            pltpu.VMEM((2 * grp, chunk, HIDDEN), jnp.float32),
            pltpu.SemaphoreType.DMA,
            pltpu.SemaphoreType.DMA,
        ],
        compiler_params=pltpu.CompilerParams(use_tc_tiling_on_sc=False),
    )
    def gather_k(idx_hbm, dst_hbm, table_hbm, out_hbm, idx_v, dst_v, rows_v, gsem, osem):
        wid = lax.axis_index("s") * info.num_cores + lax.axis_index("c")
        cbase = wid * chunks_per_w
        pltpu.sync_copy(idx_hbm.at[pl.ds(cbase, chunks_per_w)], idx_v)
        pltpu.sync_copy(dst_hbm.at[pl.ds(cbase, chunks_per_w)], dst_v)

        def fire_group(g, sb):
            for i in range(grp):
                pltpu.async_copy(
                    table_hbm.at[idx_v.at[g * grp + i]], rows_v.at[sb + i], gsem
                )

        def drain(sem, n):
            for _ in range(n):
                pltpu.make_async_copy(
                    table_hbm.at[pl.ds(0, chunk)], rows_v.at[0], sem
                ).wait()

        fire_group(0, 0)

        def body(g, carry):
            sb = (g % 2) * grp
            nsb = grp - sb
            drain(gsem, grp)

            @pl.when(g >= 1)
            def _():
                drain(osem, grp)

            @pl.when(g + 1 < n_groups)
            def _():
                fire_group(g + 1, nsb)

            for i in range(grp):
                pltpu.async_copy(
                    rows_v.at[sb + i],
                    out_hbm.at[dst_v.at[g * grp + i]],
                    osem,
                )
            return carry

        lax.fori_loop(0, n_groups, body, 0)
        drain(osem, grp)

    return gather_k(idx2d, dst2d, table)


def _project(g2, W, b, bsz, npair):
    bm = 2048
    nb = bsz // bm

    def mm_k(x_ref, w_ref, b_ref, o_ref):
        t = pl.program_id(1)

        @pl.when(t == 0)
        def _():
            o_ref[...] = jnp.broadcast_to(b_ref[...], o_ref.shape)

        o_ref[...] += jnp.dot(
            x_ref[...], w_ref[...], preferred_element_type=jnp.float32
        )

    return pl.pallas_call(
        mm_k,
        grid=(nb, npair),
        in_specs=[
            pl.BlockSpec((bm, 2 * HIDDEN), lambda i, t: (t * nb + i, 0)),
            pl.BlockSpec((2 * HIDDEN, HIDDEN), lambda i, t: (t, 0)),
            pl.BlockSpec((1, HIDDEN), lambda i, t: (0, 0)),
        ],
        out_specs=pl.BlockSpec((bm, HIDDEN), lambda i, t: (i, 0)),
        out_shape=jax.ShapeDtypeStruct((bsz, HIDDEN), jnp.float32),
    )(g2, W, b.reshape(1, HIDDEN))


def kernel(int_vals, table, W, b):
    bsz, s = int_vals.shape
    npair = s // 2
    packed = _pack_table(table.T)
    table_lin = packed.reshape(2 * packed.shape[0], HIDDEN)
    idx2d = _remap_indices(int_vals).reshape(-1, CHUNK)
    bb = jnp.arange(bsz, dtype=jnp.int32)[:, None]
    ss = jnp.arange(s, dtype=jnp.int32)[None, :]
    dst2d = (2 * ((ss // 2) * bsz + bb) + ss % 2).reshape(-1, CHUNK)
    gathered = _gather_rows(idx2d, dst2d, table_lin)
    g2 = gathered.reshape(npair * bsz, 2 * HIDDEN)
    return _project(g2, W, b, bsz, npair)
